# Initial kernel scaffold; baseline (speedup 1.0000x reference)
import jax
import jax.numpy as jnp
from jax.experimental import pallas as pl


def kernel(x, noise_init, noise_final, anneal_steps, gate_W, gate_b, W1, b1, W2, b2):
    raise NotImplementedError("write your pallas kernel here")


# speedup vs baseline: 3.3186x; 3.3186x over previous
import functools

import jax
import jax.numpy as jnp
from jax import lax
from jax.experimental import pallas as pl
from jax.experimental.pallas import tpu as pltpu
from jax.experimental.pallas import tpu_sc as plsc

D_MODEL = 1024
D_HIDDEN = 4096
N_EXPERTS = 8
TOP_K = 2
AUX_COEF = 0.01
N_TOK = 2 * 2048
CAP = int(1.25 * N_TOK / N_EXPERTS)
DUMP = N_EXPERTS * CAP
NROWS = DUMP + 8

NC, NS = 2, 16
NW = NC * NS
TOK_PER_W = N_TOK // NW
CHUNK = 32
HBLK = 512


def _gate_body(x_ref, gw_ref, gb_ref, gated_ref, e0_ref, e1_ref):
    xb = x_ref[...]
    logits = lax.dot_general(xb, gw_ref[...], (((1,), (1,)), ((), ())),
                             preferred_element_type=jnp.float32) + gb_ref[...]
    v0 = jnp.max(logits, axis=1, keepdims=True)
    p = jnp.exp(logits - v0)
    p = p / jnp.sum(p, axis=1, keepdims=True)
    iot = lax.broadcasted_iota(jnp.int32, logits.shape, 1)
    e0 = jnp.min(jnp.where(logits == v0, iot, 127), axis=1, keepdims=True)
    neg = jnp.where(iot == e0, -jnp.inf, logits)
    v1 = jnp.max(neg, axis=1, keepdims=True)
    e1 = jnp.min(jnp.where((logits == v1) & (iot != e0), iot, 127),
                 axis=1, keepdims=True)
    maskb = (iot == e0) | (iot == e1)
    gated_ref[...] = jnp.where(maskb, p, 0.0)
    e0_ref[...] = e0
    e1_ref[...] = e1


def _gating(flat, gate_W, gate_b):
    tb = 2048
    grid = (N_TOK // tb,)
    return pl.pallas_call(
        _gate_body,
        grid=grid,
        in_specs=[
            pl.BlockSpec((tb, D_MODEL), lambda i: (i, 0)),
            pl.BlockSpec((N_EXPERTS, D_MODEL), lambda i: (0, 0)),
            pl.BlockSpec((1, N_EXPERTS), lambda i: (0, 0)),
        ],
        out_specs=[
            pl.BlockSpec((tb, N_EXPERTS), lambda i: (i, 0)),
            pl.BlockSpec((tb, 1), lambda i: (i, 0)),
            pl.BlockSpec((tb, 1), lambda i: (i, 0)),
        ],
        out_shape=[
            jax.ShapeDtypeStruct((N_TOK, N_EXPERTS), jnp.float32),
            jax.ShapeDtypeStruct((N_TOK, 1), jnp.int32),
            jax.ShapeDtypeStruct((N_TOK, 1), jnp.int32),
        ],
    )(flat, gate_W, gate_b.reshape(1, N_EXPERTS))


def _route_body(gated_ref, e0_ref, e1_ref,
                d0_ref, d1_ref, g0_ref, g1_ref, c0_ref, c1_ref, aux_ref):
    g = gated_ref[...]
    gbits = lax.bitcast_convert_type(g, jnp.int32)

    def bis(_, carry):
        lo, hi = carry
        mid = lo + (hi - lo) // 2
        cnt = jnp.sum((gbits >= mid).astype(jnp.int32), axis=0, keepdims=True)
        pred = cnt >= CAP
        return jnp.where(pred, mid, lo), jnp.where(pred, hi, mid)

    lo0 = jnp.zeros((1, N_EXPERTS), jnp.int32)
    hi0 = jnp.full((1, N_EXPERTS), 0x7F800000, jnp.int32)
    thresh_bits, _ = lax.fori_loop(0, 31, bis, (lo0, hi0))

    keep = gbits >= thresh_bits
    gc = jnp.where(keep, g, 0.0)
    denom = jnp.sum(gc, axis=1, keepdims=True) + 1e-9
    gn = gc / denom
    routed = gc > 0.0
    km = routed.astype(jnp.int32)

    s = km
    sh = 1
    while sh < N_TOK:
        z = jnp.zeros((sh, N_EXPERTS), jnp.int32)
        s = s + jnp.concatenate([z, s[: N_TOK - sh]], axis=0)
        sh *= 2
    pos = s - km

    iot = lax.broadcasted_iota(jnp.int32, (N_TOK, N_EXPERTS), 1)

    def pick(e_col):
        oh = iot == e_col
        p_ = jnp.sum(jnp.where(oh, pos, 0), axis=1, keepdims=True)
        kept = jnp.sum(jnp.where(oh & routed, 1, 0), axis=1, keepdims=True) > 0
        c = jnp.sum(jnp.where(oh, gn, 0.0), axis=1, keepdims=True)
        slot = e_col * CAP + p_
        d = jnp.where(kept, slot, DUMP)
        g_ = jnp.where(kept, slot, 0)
        return d, g_, c

    d0, g0, c0 = pick(e0_ref[...])
    d1, g1, c1 = pick(e1_ref[...])
    d0_ref[...] = d0
    d1_ref[...] = d1
    g0_ref[...] = g0
    g1_ref[...] = g1
    c0_ref[...] = c0
    c1_ref[...] = c1

    imp = jnp.sum(gn, axis=0) / N_TOK
    loadv = jnp.sum(routed.astype(jnp.float32), axis=0) / N_TOK
    auxval = 0.5 * AUX_COEF * N_EXPERTS * (
        jnp.sum(imp * imp) + jnp.sum(loadv * loadv))
    aux_ref[...] = jnp.reshape(auxval, (1, 1))


def _routing(gated, e0, e1):
    return pl.pallas_call(
        _route_body,
        out_shape=[
            jax.ShapeDtypeStruct((N_TOK, 1), jnp.int32),
            jax.ShapeDtypeStruct((N_TOK, 1), jnp.int32),
            jax.ShapeDtypeStruct((N_TOK, 1), jnp.int32),
            jax.ShapeDtypeStruct((N_TOK, 1), jnp.int32),
            jax.ShapeDtypeStruct((N_TOK, 1), jnp.float32),
            jax.ShapeDtypeStruct((N_TOK, 1), jnp.float32),
            jax.ShapeDtypeStruct((1, 1), jnp.float32),
        ],
    )(gated, e0, e1)


def _disp_body(x_hbm, d0_hbm, d1_hbm, xd_hbm, idx0_v, idx1_v, rows_v, sem):
    wid = lax.axis_index("s") * NC + lax.axis_index("c")
    for ci in range(TOK_PER_W // CHUNK):
        base = wid * TOK_PER_W + ci * CHUNK
        pltpu.sync_copy(d0_hbm.at[pl.ds(base, CHUNK)], idx0_v)
        pltpu.sync_copy(d1_hbm.at[pl.ds(base, CHUNK)], idx1_v)
        pltpu.sync_copy(x_hbm.at[pl.ds(base, CHUNK)], rows_v)
        pltpu.async_copy(rows_v, xd_hbm.at[idx0_v], sem).wait()
        pltpu.async_copy(rows_v, xd_hbm.at[idx1_v], sem).wait()


def _dispatch(flat, d0, d1):
    mesh = plsc.VectorSubcoreMesh(core_axis_name="c", subcore_axis_name="s",
                                  num_cores=NC, num_subcores=NS)
    return pl.kernel(
        _disp_body,
        out_type=jax.ShapeDtypeStruct((NROWS, D_MODEL), jnp.float32),
        mesh=mesh,
        scratch_types=[
            pltpu.VMEM((CHUNK,), jnp.int32),
            pltpu.VMEM((CHUNK,), jnp.int32),
            pltpu.VMEM((CHUNK, D_MODEL), jnp.float32),
            pltpu.SemaphoreType.DMA,
        ],
    )(flat, d0, d1)


def _ffn_body(xd_ref, w1_ref, b1_ref, w2_ref, b2_ref, y_ref):
    h = pl.program_id(1)
    xb = xd_ref[0]
    hpre = lax.dot_general(xb, w1_ref[0], (((1,), (0,)), ((), ())),
                           preferred_element_type=jnp.float32) + b1_ref[0]
    hact = 0.5 * hpre * (1.0 + lax.erf(hpre * 0.7071067811865476))
    yblk = lax.dot_general(hact, w2_ref[0], (((1,), (0,)), ((), ())),
                           preferred_element_type=jnp.float32)

    @pl.when(h == 0)
    def _():
        y_ref[0] = yblk + b2_ref[0]

    @pl.when(h > 0)
    def _():
        y_ref[0] = y_ref[0] + yblk


def _ffn(xd3, W1, b1, W2, b2):
    grid = (N_EXPERTS, D_HIDDEN // HBLK)
    return pl.pallas_call(
        _ffn_body,
        grid=grid,
        in_specs=[
            pl.BlockSpec((1, CAP, D_MODEL), lambda e, h: (e, 0, 0)),
            pl.BlockSpec((1, D_MODEL, HBLK), lambda e, h: (e, 0, h)),
            pl.BlockSpec((1, 1, HBLK), lambda e, h: (e, 0, h)),
            pl.BlockSpec((1, HBLK, D_MODEL), lambda e, h: (e, h, 0)),
            pl.BlockSpec((1, 1, D_MODEL), lambda e, h: (e, 0, 0)),
        ],
        out_specs=pl.BlockSpec((1, CAP, D_MODEL), lambda e, h: (e, 0, 0)),
        out_shape=jax.ShapeDtypeStruct((N_EXPERTS, CAP, D_MODEL), jnp.float32),
        compiler_params=pltpu.CompilerParams(
            dimension_semantics=("parallel", "arbitrary")),
    )(xd3, W1, b1.reshape(N_EXPERTS, 1, D_HIDDEN), W2,
      b2.reshape(N_EXPERTS, 1, D_MODEL))


def _comb_body(y_hbm, d0_hbm, d1_hbm, y0_hbm, y1_hbm, idx_v, rows_v, sem):
    wid = lax.axis_index("s") * NC + lax.axis_index("c")
    for ci in range(TOK_PER_W // CHUNK):
        base = wid * TOK_PER_W + ci * CHUNK
        pltpu.sync_copy(d0_hbm.at[pl.ds(base, CHUNK)], idx_v)
        pltpu.async_copy(y_hbm.at[idx_v], rows_v, sem).wait()
        pltpu.sync_copy(rows_v, y0_hbm.at[pl.ds(base, CHUNK)])
        pltpu.sync_copy(d1_hbm.at[pl.ds(base, CHUNK)], idx_v)
        pltpu.async_copy(y_hbm.at[idx_v], rows_v, sem).wait()
        pltpu.sync_copy(rows_v, y1_hbm.at[pl.ds(base, CHUNK)])


def _combine_gather(yrows, d0, d1):
    mesh = plsc.VectorSubcoreMesh(core_axis_name="c", subcore_axis_name="s",
                                  num_cores=NC, num_subcores=NS)
    return pl.kernel(
        _comb_body,
        out_type=[
            jax.ShapeDtypeStruct((N_TOK, D_MODEL), jnp.float32),
            jax.ShapeDtypeStruct((N_TOK, D_MODEL), jnp.float32),
        ],
        mesh=mesh,
        scratch_types=[
            pltpu.VMEM((CHUNK,), jnp.int32),
            pltpu.VMEM((CHUNK, D_MODEL), jnp.float32),
            pltpu.SemaphoreType.DMA,
        ],
    )(yrows, d0, d1)


def _wsum_body(y0_ref, y1_ref, c0_ref, c1_ref, o_ref):
    c0 = c0_ref[...]
    c1 = c1_ref[...]
    t0 = jnp.where(c0 > 0.0, c0 * y0_ref[...], 0.0)
    t1 = jnp.where(c1 > 0.0, c1 * y1_ref[...], 0.0)
    o_ref[...] = t0 + t1


def _weighted_sum(y0, y1, c0, c1):
    tb = 1024
    return pl.pallas_call(
        _wsum_body,
        grid=(N_TOK // tb,),
        in_specs=[
            pl.BlockSpec((tb, D_MODEL), lambda i: (i, 0)),
            pl.BlockSpec((tb, D_MODEL), lambda i: (i, 0)),
            pl.BlockSpec((tb, 1), lambda i: (i, 0)),
            pl.BlockSpec((tb, 1), lambda i: (i, 0)),
        ],
        out_specs=pl.BlockSpec((tb, D_MODEL), lambda i: (i, 0)),
        out_shape=jax.ShapeDtypeStruct((N_TOK, D_MODEL), jnp.float32),
    )(y0, y1, c0, c1)


def kernel(x, noise_init, noise_final, anneal_steps, gate_W, gate_b,
           W1, b1, W2, b2):
    del noise_init, noise_final, anneal_steps
    Bb, Ll, D = x.shape
    flat = x.reshape(N_TOK, D)

    gated, e0, e1 = _gating(flat, gate_W, gate_b)
    d0, d1, g0, g1, c0, c1, aux = _routing(gated, e0, e1)

    xd = _dispatch(flat, d0.reshape(N_TOK), d1.reshape(N_TOK))
    xd3 = xd[:DUMP].reshape(N_EXPERTS, CAP, D_MODEL)
    y3 = _ffn(xd3, W1, b1, W2, b2)
    yrows = y3.reshape(DUMP, D_MODEL)
    y0, y1 = _combine_gather(yrows, g0.reshape(N_TOK), g1.reshape(N_TOK))
    out = _weighted_sum(y0, y1, c0, c1).reshape(Bb, Ll, D)
    return out, aux[0, 0]


# speedup vs baseline: 3.3894x; 1.0213x over previous
import functools

import jax
import jax.numpy as jnp
from jax import lax
from jax.experimental import pallas as pl
from jax.experimental.pallas import tpu as pltpu
from jax.experimental.pallas import tpu_sc as plsc

D_MODEL = 1024
D_HIDDEN = 4096
N_EXPERTS = 8
TOP_K = 2
AUX_COEF = 0.01
N_TOK = 2 * 2048
CAP = int(1.25 * N_TOK / N_EXPERTS)
DUMP = N_EXPERTS * CAP
NROWS = DUMP + 8

NC, NS = 2, 16
NW = NC * NS
TOK_PER_W = N_TOK // NW
CHUNK = 32
HBLK = 512


def _gate_body(x_ref, gw_ref, gb_ref, gated_ref, e0_ref, e1_ref):
    xb = x_ref[...]
    logits = lax.dot_general(xb, gw_ref[...], (((1,), (1,)), ((), ())),
                             preferred_element_type=jnp.float32) + gb_ref[...]
    v0 = jnp.max(logits, axis=1, keepdims=True)
    p = jnp.exp(logits - v0)
    p = p / jnp.sum(p, axis=1, keepdims=True)
    iot = lax.broadcasted_iota(jnp.int32, logits.shape, 1)
    e0 = jnp.min(jnp.where(logits == v0, iot, 127), axis=1, keepdims=True)
    neg = jnp.where(iot == e0, -jnp.inf, logits)
    v1 = jnp.max(neg, axis=1, keepdims=True)
    e1 = jnp.min(jnp.where((logits == v1) & (iot != e0), iot, 127),
                 axis=1, keepdims=True)
    maskb = (iot == e0) | (iot == e1)
    gated_ref[...] = jnp.where(maskb, p, 0.0)
    e0_ref[...] = e0
    e1_ref[...] = e1


def _gating(flat, gate_W, gate_b):
    tb = 2048
    grid = (N_TOK // tb,)
    return pl.pallas_call(
        _gate_body,
        grid=grid,
        in_specs=[
            pl.BlockSpec((tb, D_MODEL), lambda i: (i, 0)),
            pl.BlockSpec((N_EXPERTS, D_MODEL), lambda i: (0, 0)),
            pl.BlockSpec((1, N_EXPERTS), lambda i: (0, 0)),
        ],
        out_specs=[
            pl.BlockSpec((tb, N_EXPERTS), lambda i: (i, 0)),
            pl.BlockSpec((tb, 1), lambda i: (i, 0)),
            pl.BlockSpec((tb, 1), lambda i: (i, 0)),
        ],
        out_shape=[
            jax.ShapeDtypeStruct((N_TOK, N_EXPERTS), jnp.float32),
            jax.ShapeDtypeStruct((N_TOK, 1), jnp.int32),
            jax.ShapeDtypeStruct((N_TOK, 1), jnp.int32),
        ],
    )(flat, gate_W, gate_b.reshape(1, N_EXPERTS))


def _route_body(gated_ref, e0_ref, e1_ref,
                d0_ref, d1_ref, g0_ref, g1_ref, c0_ref, c1_ref, aux_ref):
    g = gated_ref[...]
    gbits = lax.bitcast_convert_type(g, jnp.int32)

    def bis(_, carry):
        lo, hi = carry
        mid = lo + (hi - lo) // 2
        cnt = jnp.sum((gbits >= mid).astype(jnp.int32), axis=0, keepdims=True)
        pred = cnt >= CAP
        return jnp.where(pred, mid, lo), jnp.where(pred, hi, mid)

    lo0 = jnp.zeros((1, N_EXPERTS), jnp.int32)
    hi0 = jnp.full((1, N_EXPERTS), 0x7F800000, jnp.int32)
    thresh_bits, _ = lax.fori_loop(0, 31, bis, (lo0, hi0))

    keep = gbits >= thresh_bits
    gc = jnp.where(keep, g, 0.0)
    denom = jnp.sum(gc, axis=1, keepdims=True) + 1e-9
    gn = gc / denom
    routed = gc > 0.0
    km = routed.astype(jnp.int32)

    s = km
    sh = 1
    while sh < N_TOK:
        z = jnp.zeros((sh, N_EXPERTS), jnp.int32)
        s = s + jnp.concatenate([z, s[: N_TOK - sh]], axis=0)
        sh *= 2
    pos = s - km

    iot = lax.broadcasted_iota(jnp.int32, (N_TOK, N_EXPERTS), 1)

    def pick(e_col):
        oh = iot == e_col
        p_ = jnp.sum(jnp.where(oh, pos, 0), axis=1, keepdims=True)
        kept = jnp.sum(jnp.where(oh & routed, 1, 0), axis=1, keepdims=True) > 0
        c = jnp.sum(jnp.where(oh, gn, 0.0), axis=1, keepdims=True)
        slot = e_col * CAP + p_
        d = jnp.where(kept, slot, DUMP)
        g_ = jnp.where(kept, slot, 0)
        return d, g_, c

    d0, g0, c0 = pick(e0_ref[...])
    d1, g1, c1 = pick(e1_ref[...])
    d0_ref[...] = d0
    d1_ref[...] = d1
    g0_ref[...] = g0
    g1_ref[...] = g1
    c0_ref[...] = c0
    c1_ref[...] = c1

    imp = jnp.sum(gn, axis=0) / N_TOK
    loadv = jnp.sum(routed.astype(jnp.float32), axis=0) / N_TOK
    auxval = 0.5 * AUX_COEF * N_EXPERTS * (
        jnp.sum(imp * imp) + jnp.sum(loadv * loadv))
    aux_ref[...] = jnp.reshape(auxval, (1, 1))


def _routing(gated, e0, e1):
    return pl.pallas_call(
        _route_body,
        out_shape=[
            jax.ShapeDtypeStruct((N_TOK, 1), jnp.int32),
            jax.ShapeDtypeStruct((N_TOK, 1), jnp.int32),
            jax.ShapeDtypeStruct((N_TOK, 1), jnp.int32),
            jax.ShapeDtypeStruct((N_TOK, 1), jnp.int32),
            jax.ShapeDtypeStruct((N_TOK, 1), jnp.float32),
            jax.ShapeDtypeStruct((N_TOK, 1), jnp.float32),
            jax.ShapeDtypeStruct((1, 1), jnp.float32),
        ],
    )(gated, e0, e1)


NCHUNK = TOK_PER_W // CHUNK
NBUF = 3


def _disp_body(x_hbm, d0_hbm, d1_hbm, xd_hbm, idx0_v, idx1_v, rows_v,
               isem, lsem, ssem):
    wid = lax.axis_index("s") * NC + lax.axis_index("c")
    base = wid * TOK_PER_W
    idescs = []
    for ci in range(NCHUNK):
        idescs.append(pltpu.async_copy(
            d0_hbm.at[pl.ds(base + ci * CHUNK, CHUNK)], idx0_v.at[ci], isem))
        idescs.append(pltpu.async_copy(
            d1_hbm.at[pl.ds(base + ci * CHUNK, CHUNK)], idx1_v.at[ci], isem))
    for d in idescs:
        d.wait()

    loads = [None] * NCHUNK
    scats = [None] * NCHUNK

    def start_load(ci):
        loads[ci] = pltpu.async_copy(
            x_hbm.at[pl.ds(base + ci * CHUNK, CHUNK)], rows_v.at[ci % NBUF],
            lsem)

    for ci in range(min(NBUF, NCHUNK)):
        start_load(ci)
    for ci in range(NCHUNK):
        loads[ci].wait()
        scats[ci] = (
            pltpu.async_copy(rows_v.at[ci % NBUF], xd_hbm.at[idx0_v.at[ci]],
                             ssem),
            pltpu.async_copy(rows_v.at[ci % NBUF], xd_hbm.at[idx1_v.at[ci]],
                             ssem),
        )
        j = ci + NBUF
        if j < NCHUNK:
            for d in scats[j - NBUF]:
                d.wait()
            scats[j - NBUF] = None
            start_load(j)
    for pair in scats:
        if pair is not None:
            for d in pair:
                d.wait()


def _dispatch(flat, d0, d1):
    mesh = plsc.VectorSubcoreMesh(core_axis_name="c", subcore_axis_name="s",
                                  num_cores=NC, num_subcores=NS)
    return pl.kernel(
        _disp_body,
        out_type=jax.ShapeDtypeStruct((NROWS, D_MODEL), jnp.float32),
        mesh=mesh,
        scratch_types=[
            pltpu.VMEM((NCHUNK, CHUNK), jnp.int32),
            pltpu.VMEM((NCHUNK, CHUNK), jnp.int32),
            pltpu.VMEM((NBUF, CHUNK, D_MODEL), jnp.float32),
            pltpu.SemaphoreType.DMA,
            pltpu.SemaphoreType.DMA,
            pltpu.SemaphoreType.DMA,
        ],
    )(flat, d0, d1)


def _ffn_body(xd_ref, w1_ref, b1_ref, w2_ref, b2_ref, y_ref):
    h = pl.program_id(1)
    xb = xd_ref[0]
    hpre = lax.dot_general(xb, w1_ref[0], (((1,), (0,)), ((), ())),
                           preferred_element_type=jnp.float32) + b1_ref[0]
    hact = 0.5 * hpre * (1.0 + lax.erf(hpre * 0.7071067811865476))
    yblk = lax.dot_general(hact, w2_ref[0], (((1,), (0,)), ((), ())),
                           preferred_element_type=jnp.float32)

    @pl.when(h == 0)
    def _():
        y_ref[0] = yblk + b2_ref[0]

    @pl.when(h > 0)
    def _():
        y_ref[0] = y_ref[0] + yblk


def _ffn(xd3, W1, b1, W2, b2):
    grid = (N_EXPERTS, D_HIDDEN // HBLK)
    return pl.pallas_call(
        _ffn_body,
        grid=grid,
        in_specs=[
            pl.BlockSpec((1, CAP, D_MODEL), lambda e, h: (e, 0, 0)),
            pl.BlockSpec((1, D_MODEL, HBLK), lambda e, h: (e, 0, h)),
            pl.BlockSpec((1, 1, HBLK), lambda e, h: (e, 0, h)),
            pl.BlockSpec((1, HBLK, D_MODEL), lambda e, h: (e, h, 0)),
            pl.BlockSpec((1, 1, D_MODEL), lambda e, h: (e, 0, 0)),
        ],
        out_specs=pl.BlockSpec((1, CAP, D_MODEL), lambda e, h: (e, 0, 0)),
        out_shape=jax.ShapeDtypeStruct((N_EXPERTS, CAP, D_MODEL), jnp.float32),
        compiler_params=pltpu.CompilerParams(
            dimension_semantics=("parallel", "arbitrary")),
    )(xd3, W1, b1.reshape(N_EXPERTS, 1, D_HIDDEN), W2,
      b2.reshape(N_EXPERTS, 1, D_MODEL))


def _comb_body(y_hbm, d0_hbm, d1_hbm, y0_hbm, y1_hbm, gidx_v, rows_v,
               isem, gsem, wsem):
    wid = lax.axis_index("s") * NC + lax.axis_index("c")
    base = wid * TOK_PER_W
    ntask = 2 * NCHUNK
    idescs = []
    for ci in range(NCHUNK):
        idescs.append(pltpu.async_copy(
            d0_hbm.at[pl.ds(base + ci * CHUNK, CHUNK)], gidx_v.at[2 * ci],
            isem))
        idescs.append(pltpu.async_copy(
            d1_hbm.at[pl.ds(base + ci * CHUNK, CHUNK)], gidx_v.at[2 * ci + 1],
            isem))
    for d in idescs:
        d.wait()

    gath = [None] * ntask
    stor = [None] * ntask
    outs = [y0_hbm, y1_hbm]

    def start_gather(t):
        gath[t] = pltpu.async_copy(y_hbm.at[gidx_v.at[t]],
                                   rows_v.at[t % NBUF], gsem)

    for t in range(min(NBUF, ntask)):
        start_gather(t)
    for t in range(ntask):
        gath[t].wait()
        ci, k = t // 2, t % 2
        stor[t] = pltpu.async_copy(
            rows_v.at[t % NBUF],
            outs[k].at[pl.ds(base + ci * CHUNK, CHUNK)], wsem)
        j = t + NBUF
        if j < ntask:
            stor[j - NBUF].wait()
            stor[j - NBUF] = None
            start_gather(j)
    for d in stor:
        if d is not None:
            d.wait()


def _combine_gather(yrows, d0, d1):
    mesh = plsc.VectorSubcoreMesh(core_axis_name="c", subcore_axis_name="s",
                                  num_cores=NC, num_subcores=NS)
    return pl.kernel(
        _comb_body,
        out_type=[
            jax.ShapeDtypeStruct((N_TOK, D_MODEL), jnp.float32),
            jax.ShapeDtypeStruct((N_TOK, D_MODEL), jnp.float32),
        ],
        mesh=mesh,
        scratch_types=[
            pltpu.VMEM((2 * NCHUNK, CHUNK), jnp.int32),
            pltpu.VMEM((NBUF, CHUNK, D_MODEL), jnp.float32),
            pltpu.SemaphoreType.DMA,
            pltpu.SemaphoreType.DMA,
            pltpu.SemaphoreType.DMA,
        ],
    )(yrows, d0, d1)


def _wsum_body(y0_ref, y1_ref, c0_ref, c1_ref, o_ref):
    c0 = c0_ref[...]
    c1 = c1_ref[...]
    t0 = jnp.where(c0 > 0.0, c0 * y0_ref[...], 0.0)
    t1 = jnp.where(c1 > 0.0, c1 * y1_ref[...], 0.0)
    o_ref[...] = t0 + t1


def _weighted_sum(y0, y1, c0, c1):
    tb = 1024
    return pl.pallas_call(
        _wsum_body,
        grid=(N_TOK // tb,),
        in_specs=[
            pl.BlockSpec((tb, D_MODEL), lambda i: (i, 0)),
            pl.BlockSpec((tb, D_MODEL), lambda i: (i, 0)),
            pl.BlockSpec((tb, 1), lambda i: (i, 0)),
            pl.BlockSpec((tb, 1), lambda i: (i, 0)),
        ],
        out_specs=pl.BlockSpec((tb, D_MODEL), lambda i: (i, 0)),
        out_shape=jax.ShapeDtypeStruct((N_TOK, D_MODEL), jnp.float32),
    )(y0, y1, c0, c1)


def kernel(x, noise_init, noise_final, anneal_steps, gate_W, gate_b,
           W1, b1, W2, b2):
    del noise_init, noise_final, anneal_steps
    Bb, Ll, D = x.shape
    flat = x.reshape(N_TOK, D)

    gated, e0, e1 = _gating(flat, gate_W, gate_b)
    d0, d1, g0, g1, c0, c1, aux = _routing(gated, e0, e1)

    xd = _dispatch(flat, d0.reshape(N_TOK), d1.reshape(N_TOK))
    xd3 = xd[:DUMP].reshape(N_EXPERTS, CAP, D_MODEL)
    y3 = _ffn(xd3, W1, b1, W2, b2)
    yrows = y3.reshape(DUMP, D_MODEL)
    y0, y1 = _combine_gather(yrows, g0.reshape(N_TOK), g1.reshape(N_TOK))
    out = _weighted_sum(y0, y1, c0, c1).reshape(Bb, Ll, D)
    return out, aux[0, 0]
